# baseline (speedup 1.0000x reference)
import functools

import jax
import jax.numpy as jnp
from jax import lax
from jax.experimental import pallas as pl
from jax.experimental.pallas import tpu as pltpu

_EPS = 1e-5
_NEG = float(jnp.finfo(jnp.float32).min)


def _shift_lanes(a, k, fill):
    if k == 0:
        return a
    c, l = a.shape
    f = jnp.full((c, abs(k)), fill, a.dtype)
    if k > 0:
        return jnp.concatenate([a[:, k:], f], axis=1)
    return jnp.concatenate([f, a[:, :l + k]], axis=1)


def _wshift(a, dw, w, fill):
    s = _shift_lanes(a, dw, fill)
    if dw == 0:
        return s
    wi = lax.rem(lax.broadcasted_iota(jnp.int32, a.shape, 1), jnp.int32(w))
    if dw > 0:
        valid = wi < (w - dw)
    else:
        valid = wi >= (-dw)
    return jnp.where(valid, s, fill)


def _accum_stats(st_ref, vals_list, ones_ref, pid):
    local = None
    for vals_bf in vals_list:
        both = jnp.concatenate([vals_bf, vals_bf * vals_bf], axis=0)
        d = jnp.dot(both, ones_ref[...], preferred_element_type=jnp.float32)
        local = d if local is None else local + d

    @pl.when(pid == 0)
    def _():
        st_ref[...] = local

    @pl.when(pid > 0)
    def _():
        st_ref[...] = st_ref[...] + local


def _fold(s_rows, q_rows, g_col, b_col, count):
    mean = s_rows[:, 0:1] / count
    var = q_rows[:, 0:1] / count - mean * mean
    scale = g_col * lax.rsqrt(var + _EPS)
    shift = b_col - mean * scale
    c = scale.shape[0]
    return (jnp.broadcast_to(scale, (c, 128)),
            jnp.broadcast_to(shift, (c, 128)))


def _col(*rows):
    return jnp.transpose(jnp.concatenate(rows, axis=1))


def _mega_body(x_ref, w1_ref, w2a_ref, w3a_ref, wp_ref, w3_ref, w5_ref,
               ones_ref,
               g1_ref, g4_ref, g2a_ref, g3a_ref, g2b_ref, g3b_ref,
               h1_ref, h4_ref, h2a_ref, h3a_ref, h2b_ref, h3b_ref,
               out_ref,
               keep_s, mid_s, acc2_s, st1_s, st2_s, scm_s, shm_s,
               sca_s, sha_s, wxb_s, wpb_s, w3b_s, w5b_s, *,
               n_imgs, blk, h, w, out1, red3, red5, out3):
    phase = pl.program_id(0)
    i = pl.program_id(1)
    count = float(n_imgs * h * w)
    cmid = red3 + red5
    ckeep = keep_s.shape[1]
    c1 = ckeep + cmid

    @pl.when(jnp.logical_and(phase == 0, i == 0))
    def _():
        wxb_s[...] = jnp.concatenate(
            [jnp.transpose(w1_ref[...]), jnp.transpose(w2a_ref[...]),
             jnp.transpose(w3a_ref[...])], axis=0).astype(jnp.bfloat16)
        wpb_s[...] = jnp.transpose(wp_ref[...]).astype(jnp.bfloat16)
        w3b_s[...] = jnp.transpose(w3_ref[...]).astype(jnp.bfloat16)
        w5b_s[...] = jnp.transpose(w5_ref[...]).astype(jnp.bfloat16)

    @pl.when(phase == 0)
    def _():
        vals = []
        for j in range(blk):
            xx = x_ref[j]
            y = jnp.dot(wxb_s[...], xx.astype(jnp.bfloat16),
                        preferred_element_type=jnp.float32)

            rowm = jnp.maximum(xx, jnp.maximum(_wshift(xx, 1, w, _NEG),
                                               _wshift(xx, -1, w, _NEG)))
            pooled = jnp.maximum(
                rowm, jnp.maximum(_shift_lanes(rowm, w, _NEG),
                                  _shift_lanes(rowm, -w, _NEG)))
            yp = jnp.dot(wpb_s[...], pooled.astype(jnp.bfloat16),
                         preferred_element_type=jnp.float32)

            keep_bf = jnp.concatenate([y[:out1], yp],
                                      axis=0).astype(jnp.bfloat16)
            mid_bf = y[out1:].astype(jnp.bfloat16)
            keep_s[blk * i + j] = keep_bf
            mid_s[blk * i + j] = mid_bf
            vals.append(jnp.concatenate([keep_bf, mid_bf], axis=0))
        _accum_stats(st1_s, vals, ones_ref, i)

    @pl.when(jnp.logical_and(phase == 1, i == 0))
    def _():
        sc, sh = _fold(st1_s[ckeep:c1], st1_s[c1 + ckeep:2 * c1],
                       _col(g2a_ref[...], g3a_ref[...]),
                       _col(h2a_ref[...], h3a_ref[...]), count)
        scm_s[...] = sc
        shm_s[...] = sh

    @pl.when(phase == 1)
    def _():
        vals = []
        for j in range(blk):
            mid = mid_s[blk * i + j].astype(jnp.float32)
            a = jnp.maximum(mid * scm_s[:, 0:1] + shm_s[:, 0:1], 0.0)
            r3 = a[:red3]
            r5 = a[red3:]

            xw3 = jnp.concatenate(
                [_wshift(r3, dw, w, 0.0) for dw in (-1, 0, 1)],
                axis=0).astype(jnp.bfloat16)
            y3 = _shift_lanes(
                jnp.dot(w3b_s[:, :3 * red3], xw3,
                        preferred_element_type=jnp.float32), -w, 0.0)
            y3 += jnp.dot(w3b_s[:, 3 * red3:6 * red3], xw3,
                          preferred_element_type=jnp.float32)
            y3 += _shift_lanes(
                jnp.dot(w3b_s[:, 6 * red3:], xw3,
                        preferred_element_type=jnp.float32), w, 0.0)

            xw5 = jnp.concatenate(
                [_wshift(r5, dw, w, 0.0) for dw in (-2, -1, 0, 1, 2)],
                axis=0).astype(jnp.bfloat16)
            y5 = 0.0
            for kh in range(5):
                z = jnp.dot(w5b_s[:, kh * 5 * red5:(kh + 1) * 5 * red5],
                            xw5, preferred_element_type=jnp.float32)
                y5 = y5 + _shift_lanes(z, (kh - 2) * w, 0.0)

            out2_bf = jnp.concatenate([y3, y5], axis=0).astype(jnp.bfloat16)
            acc2_s[blk * i + j] = out2_bf
            vals.append(out2_bf)
        _accum_stats(st2_s, vals, ones_ref, i)

    @pl.when(jnp.logical_and(phase == 2, i == 0))
    def _():
        c2 = acc2_s.shape[1]
        sck, shk = _fold(st1_s[:ckeep], st1_s[c1:c1 + ckeep],
                         _col(g1_ref[...], g4_ref[...]),
                         _col(h1_ref[...], h4_ref[...]), count)
        sc2, sh2 = _fold(st2_s[:c2], st2_s[c2:],
                         _col(g2b_ref[...], g3b_ref[...]),
                         _col(h2b_ref[...], h3b_ref[...]), count)
        sca_s[...] = jnp.concatenate([sck[:out1], sc2, sck[out1:]], axis=0)
        sha_s[...] = jnp.concatenate([shk[:out1], sh2, shk[out1:]], axis=0)

    @pl.when(phase == 2)
    def _():
        for j in range(blk):
            k = keep_s[blk * i + j]
            raw = jnp.concatenate([k[:out1], acc2_s[blk * i + j], k[out1:]],
                                  axis=0).astype(jnp.float32)
            res = jnp.maximum(raw * sca_s[:, 0:1] + sha_s[:, 0:1], 0.0)
            out_ref[j] = res.astype(jnp.bfloat16)


def kernel(x, b1_w, b1_gamma, b1_beta, b2a_w, b2a_gamma, b2a_beta,
           b2b_w, b2b_gamma, b2b_beta, b3a_w, b3a_gamma, b3a_beta,
           b3b_w, b3b_gamma, b3b_beta, b4_w, b4_gamma, b4_beta):
    x = x.astype(jnp.float32)
    n, cin, h, w = x.shape
    hw = h * w
    out1 = b1_w.shape[-1]
    red3, out3 = b2a_w.shape[-1], b2b_w.shape[-1]
    red5, out5 = b3a_w.shape[-1], b3b_w.shape[-1]
    outp = b4_w.shape[-1]
    cmid = red3 + red5
    ckeep = out1 + outp
    c2 = out3 + out5
    cout = out1 + out3 + out5 + outp

    w1_r = b1_w.reshape(cin, out1)
    w2a_r = b2a_w.reshape(cin, red3)
    w3a_r = b3a_w.reshape(cin, red5)
    wp_r = b4_w.reshape(cin, outp)
    w3_r = b2b_w.reshape(9 * red3, out3)
    w5_r = b3b_w.reshape(25 * red5, out5)

    ones_bf = jnp.ones((hw, 128), jnp.bfloat16)
    gb_vecs = [v.reshape(1, -1) for v in
               (b1_gamma, b4_gamma, b2a_gamma, b3a_gamma, b2b_gamma,
                b3b_gamma, b1_beta, b4_beta, b2a_beta, b3a_beta,
                b2b_beta, b3b_beta)]

    blk = next(b for b in (8, 4, 2, 1) if n % b == 0)
    last = n // blk - 1
    out = pl.pallas_call(
        functools.partial(_mega_body, n_imgs=n, blk=blk, h=h, w=w,
                          out1=out1, red3=red3, red5=red5, out3=out3),
        grid=(3, n // blk),
        in_specs=[
            pl.BlockSpec((blk, cin, hw),
                         lambda p, i: (jnp.where(p == 0, i, last), 0, 0)),
            pl.BlockSpec((cin, out1), lambda p, i: (0, 0)),
            pl.BlockSpec((cin, red3), lambda p, i: (0, 0)),
            pl.BlockSpec((cin, red5), lambda p, i: (0, 0)),
            pl.BlockSpec((cin, outp), lambda p, i: (0, 0)),
            pl.BlockSpec((9 * red3, out3), lambda p, i: (0, 0)),
            pl.BlockSpec((25 * red5, out5), lambda p, i: (0, 0)),
            pl.BlockSpec((hw, 128), lambda p, i: (0, 0)),
        ] + [pl.BlockSpec(v.shape, lambda p, i: (0, 0)) for v in gb_vecs],
        out_specs=pl.BlockSpec(
            (blk, cout, hw),
            lambda p, i: (jnp.where(p == 2, i, 0), 0, 0)),
        out_shape=jax.ShapeDtypeStruct((n, cout, hw), jnp.bfloat16),
        scratch_shapes=[
            pltpu.VMEM((n, ckeep, hw), jnp.bfloat16),
            pltpu.VMEM((n, cmid, hw), jnp.bfloat16),
            pltpu.VMEM((n, c2, hw), jnp.bfloat16),
            pltpu.VMEM((2 * (ckeep + cmid), 128), jnp.float32),
            pltpu.VMEM((2 * c2, 128), jnp.float32),
            pltpu.VMEM((cmid, 128), jnp.float32),
            pltpu.VMEM((cmid, 128), jnp.float32),
            pltpu.VMEM((cout, 128), jnp.float32),
            pltpu.VMEM((cout, 128), jnp.float32),
            pltpu.VMEM((out1 + cmid, cin), jnp.bfloat16),
            pltpu.VMEM((outp, cin), jnp.bfloat16),
            pltpu.VMEM((out3, 9 * red3), jnp.bfloat16),
            pltpu.VMEM((out5, 25 * red5), jnp.bfloat16),
        ],
        compiler_params=pltpu.CompilerParams(
            dimension_semantics=("arbitrary", "arbitrary"),
            vmem_limit_bytes=100 * 1024 * 1024),
    )(x.reshape(n, cin, hw), w1_r, w2a_r, w3a_r, wp_r, w3_r, w5_r,
      ones_bf, *gb_vecs)
    return out.reshape(n, cout, h, w).astype(jnp.float32)


# speedup vs baseline: 1.0036x; 1.0036x over previous
import functools

import jax
import jax.numpy as jnp
from jax import lax
from jax.experimental import pallas as pl
from jax.experimental.pallas import tpu as pltpu

_EPS = 1e-5
_NEG = float(jnp.finfo(jnp.float32).min)


def _shift_lanes(a, k, fill):
    if k == 0:
        return a
    c, l = a.shape
    f = jnp.full((c, abs(k)), fill, a.dtype)
    if k > 0:
        return jnp.concatenate([a[:, k:], f], axis=1)
    return jnp.concatenate([f, a[:, :l + k]], axis=1)


def _wshift(a, dw, w, fill):
    s = _shift_lanes(a, dw, fill)
    if dw == 0:
        return s
    wi = lax.rem(lax.broadcasted_iota(jnp.int32, a.shape, 1), jnp.int32(w))
    if dw > 0:
        valid = wi < (w - dw)
    else:
        valid = wi >= (-dw)
    return jnp.where(valid, s, fill)


def _accum_stats(st_ref, vals_list, ones_ref, pid):
    local = None
    for vals_bf in vals_list:
        both = jnp.concatenate([vals_bf, vals_bf * vals_bf], axis=0)
        d = jnp.dot(both, ones_ref[...], preferred_element_type=jnp.float32)
        local = d if local is None else local + d

    @pl.when(pid == 0)
    def _():
        st_ref[...] = local

    @pl.when(pid > 0)
    def _():
        st_ref[...] = st_ref[...] + local


def _fold(s_rows, q_rows, g_col, b_col, count):
    mean = s_rows[:, 0:1] / count
    var = q_rows[:, 0:1] / count - mean * mean
    scale = g_col * lax.rsqrt(var + _EPS)
    shift = b_col - mean * scale
    c = scale.shape[0]
    return (jnp.broadcast_to(scale, (c, 128)),
            jnp.broadcast_to(shift, (c, 128)))


def _col(*rows):
    return jnp.transpose(jnp.concatenate(rows, axis=1))


def _mega_body(x_ref, w1_ref, w2a_ref, w3a_ref, wp_ref, w3_ref, w5_ref,
               ones_ref,
               g1_ref, g4_ref, g2a_ref, g3a_ref, g2b_ref, g3b_ref,
               h1_ref, h4_ref, h2a_ref, h3a_ref, h2b_ref, h3b_ref,
               out_ref,
               keep_s, mid_s, acc2_s, st1_s, st2_s, scm_s, shm_s,
               sca_s, sha_s, wxb_s, wpb_s, w3b_s, w5b_s, *,
               n_imgs, blk, h, w, out1, red3, red5, out3):
    phase = pl.program_id(0)
    i = pl.program_id(1)
    count = float(n_imgs * h * w)
    cmid = red3 + red5
    ckeep = keep_s.shape[1]
    c1 = ckeep + cmid

    @pl.when(jnp.logical_and(phase == 0, i == 0))
    def _():
        wxb_s[...] = jnp.concatenate(
            [jnp.transpose(w1_ref[...]), jnp.transpose(w2a_ref[...]),
             jnp.transpose(w3a_ref[...])], axis=0).astype(jnp.bfloat16)
        wpb_s[...] = jnp.transpose(wp_ref[...]).astype(jnp.bfloat16)
        kk3 = w3_ref.shape[0] * w3_ref.shape[1] * w3_ref.shape[2]
        kk5 = w5_ref.shape[0] * w5_ref.shape[1] * w5_ref.shape[2]
        w3b_s[...] = jnp.transpose(
            w3_ref[...].reshape(kk3, w3_ref.shape[3])).astype(jnp.bfloat16)
        w5b_s[...] = jnp.transpose(
            w5_ref[...].reshape(kk5, w5_ref.shape[3])).astype(jnp.bfloat16)

    @pl.when(phase == 0)
    def _():
        vals = []
        for j in range(blk):
            xx = x_ref[j]
            y = jnp.dot(wxb_s[...], xx.astype(jnp.bfloat16),
                        preferred_element_type=jnp.float32)

            rowm = jnp.maximum(xx, jnp.maximum(_wshift(xx, 1, w, _NEG),
                                               _wshift(xx, -1, w, _NEG)))
            pooled = jnp.maximum(
                rowm, jnp.maximum(_shift_lanes(rowm, w, _NEG),
                                  _shift_lanes(rowm, -w, _NEG)))
            yp = jnp.dot(wpb_s[...], pooled.astype(jnp.bfloat16),
                         preferred_element_type=jnp.float32)

            keep_bf = jnp.concatenate([y[:out1], yp],
                                      axis=0).astype(jnp.bfloat16)
            mid_bf = y[out1:].astype(jnp.bfloat16)
            keep_s[blk * i + j] = keep_bf
            mid_s[blk * i + j] = mid_bf
            vals.append(jnp.concatenate([keep_bf, mid_bf], axis=0))
        _accum_stats(st1_s, vals, ones_ref, i)

    @pl.when(jnp.logical_and(phase == 1, i == 0))
    def _():
        sc, sh = _fold(st1_s[ckeep:c1], st1_s[c1 + ckeep:2 * c1],
                       _col(g2a_ref[...], g3a_ref[...]),
                       _col(h2a_ref[...], h3a_ref[...]), count)
        scm_s[...] = sc
        shm_s[...] = sh

    @pl.when(phase == 1)
    def _():
        vals = []
        for j in range(blk):
            mid = mid_s[blk * i + j].astype(jnp.float32)
            a = jnp.maximum(mid * scm_s[:, 0:1] + shm_s[:, 0:1], 0.0)
            r3 = a[:red3]
            r5 = a[red3:]

            xw3 = jnp.concatenate(
                [_wshift(r3, dw, w, 0.0) for dw in (-1, 0, 1)],
                axis=0).astype(jnp.bfloat16)
            y3 = _shift_lanes(
                jnp.dot(w3b_s[:, :3 * red3], xw3,
                        preferred_element_type=jnp.float32), -w, 0.0)
            y3 += jnp.dot(w3b_s[:, 3 * red3:6 * red3], xw3,
                          preferred_element_type=jnp.float32)
            y3 += _shift_lanes(
                jnp.dot(w3b_s[:, 6 * red3:], xw3,
                        preferred_element_type=jnp.float32), w, 0.0)

            xw5 = jnp.concatenate(
                [_wshift(r5, dw, w, 0.0) for dw in (-2, -1, 0, 1, 2)],
                axis=0).astype(jnp.bfloat16)
            y5 = 0.0
            for kh in range(5):
                z = jnp.dot(w5b_s[:, kh * 5 * red5:(kh + 1) * 5 * red5],
                            xw5, preferred_element_type=jnp.float32)
                y5 = y5 + _shift_lanes(z, (kh - 2) * w, 0.0)

            out2_bf = jnp.concatenate([y3, y5], axis=0).astype(jnp.bfloat16)
            acc2_s[blk * i + j] = out2_bf
            vals.append(out2_bf)
        _accum_stats(st2_s, vals, ones_ref, i)

    @pl.when(jnp.logical_and(phase == 2, i == 0))
    def _():
        c2 = acc2_s.shape[1]
        sck, shk = _fold(st1_s[:ckeep], st1_s[c1:c1 + ckeep],
                         _col(g1_ref[...], g4_ref[...]),
                         _col(h1_ref[...], h4_ref[...]), count)
        sc2, sh2 = _fold(st2_s[:c2], st2_s[c2:],
                         _col(g2b_ref[...], g3b_ref[...]),
                         _col(h2b_ref[...], h3b_ref[...]), count)
        sca_s[...] = jnp.concatenate([sck[:out1], sc2, sck[out1:]], axis=0)
        sha_s[...] = jnp.concatenate([shk[:out1], sh2, shk[out1:]], axis=0)

    @pl.when(phase == 2)
    def _():
        for j in range(blk):
            k = keep_s[blk * i + j]
            raw = jnp.concatenate([k[:out1], acc2_s[blk * i + j], k[out1:]],
                                  axis=0).astype(jnp.float32)
            res = jnp.maximum(raw * sca_s[:, 0:1] + sha_s[:, 0:1], 0.0)
            out_ref[j] = res.astype(jnp.bfloat16)


def kernel(x, b1_w, b1_gamma, b1_beta, b2a_w, b2a_gamma, b2a_beta,
           b2b_w, b2b_gamma, b2b_beta, b3a_w, b3a_gamma, b3a_beta,
           b3b_w, b3b_gamma, b3b_beta, b4_w, b4_gamma, b4_beta):
    x = x.astype(jnp.float32)
    n, cin, h, w = x.shape
    hw = h * w
    out1 = b1_w.shape[-1]
    red3, out3 = b2a_w.shape[-1], b2b_w.shape[-1]
    red5, out5 = b3a_w.shape[-1], b3b_w.shape[-1]
    outp = b4_w.shape[-1]
    cmid = red3 + red5
    ckeep = out1 + outp
    c2 = out3 + out5
    cout = out1 + out3 + out5 + outp

    w1_r = b1_w.reshape(cin, out1)
    w2a_r = b2a_w.reshape(cin, red3)
    w3a_r = b3a_w.reshape(cin, red5)
    wp_r = b4_w.reshape(cin, outp)

    ones_bf = jnp.ones((hw, 128), jnp.bfloat16)
    gb_vecs = [v.reshape(1, -1) for v in
               (b1_gamma, b4_gamma, b2a_gamma, b3a_gamma, b2b_gamma,
                b3b_gamma, b1_beta, b4_beta, b2a_beta, b3a_beta,
                b2b_beta, b3b_beta)]

    blk = next(b for b in (8, 4, 2, 1) if n % b == 0)
    last = n // blk - 1
    out = pl.pallas_call(
        functools.partial(_mega_body, n_imgs=n, blk=blk, h=h, w=w,
                          out1=out1, red3=red3, red5=red5, out3=out3),
        grid=(3, n // blk),
        in_specs=[
            pl.BlockSpec((blk, cin, hw),
                         lambda p, i: (jnp.where(p == 0, i, last), 0, 0)),
            pl.BlockSpec((cin, out1), lambda p, i: (0, 0)),
            pl.BlockSpec((cin, red3), lambda p, i: (0, 0)),
            pl.BlockSpec((cin, red5), lambda p, i: (0, 0)),
            pl.BlockSpec((cin, outp), lambda p, i: (0, 0)),
            pl.BlockSpec((3, 3, red3, out3), lambda p, i: (0, 0, 0, 0)),
            pl.BlockSpec((5, 5, red5, out5),
                         lambda p, i: (0, 0, 0, 0)),
            pl.BlockSpec((hw, 128), lambda p, i: (0, 0)),
        ] + [pl.BlockSpec(v.shape, lambda p, i: (0, 0)) for v in gb_vecs],
        out_specs=pl.BlockSpec(
            (blk, cout, hw),
            lambda p, i: (jnp.where(p == 2, i, 0), 0, 0)),
        out_shape=jax.ShapeDtypeStruct((n, cout, hw), jnp.bfloat16),
        scratch_shapes=[
            pltpu.VMEM((n, ckeep, hw), jnp.bfloat16),
            pltpu.VMEM((n, cmid, hw), jnp.bfloat16),
            pltpu.VMEM((n, c2, hw), jnp.bfloat16),
            pltpu.VMEM((2 * (ckeep + cmid), 128), jnp.float32),
            pltpu.VMEM((2 * c2, 128), jnp.float32),
            pltpu.VMEM((cmid, 128), jnp.float32),
            pltpu.VMEM((cmid, 128), jnp.float32),
            pltpu.VMEM((cout, 128), jnp.float32),
            pltpu.VMEM((cout, 128), jnp.float32),
            pltpu.VMEM((out1 + cmid, cin), jnp.bfloat16),
            pltpu.VMEM((outp, cin), jnp.bfloat16),
            pltpu.VMEM((out3, 9 * red3), jnp.bfloat16),
            pltpu.VMEM((out5, 25 * red5), jnp.bfloat16),
        ],
        compiler_params=pltpu.CompilerParams(
            dimension_semantics=("arbitrary", "arbitrary"),
            vmem_limit_bytes=100 * 1024 * 1024),
    )(x.reshape(n, cin, hw), w1_r, w2a_r, w3a_r, wp_r, b2b_w, b3b_w,
      ones_bf, *gb_vecs)
    return out.reshape(n, cout, h, w).astype(jnp.float32)


# speedup vs baseline: 1.0124x; 1.0088x over previous
import functools

import jax
import jax.numpy as jnp
from jax import lax
from jax.experimental import pallas as pl
from jax.experimental.pallas import tpu as pltpu

_EPS = 1e-5
_NEG = float(jnp.finfo(jnp.float32).min)


def _shift_lanes(a, k, fill):
    if k == 0:
        return a
    c, l = a.shape
    f = jnp.full((c, abs(k)), fill, a.dtype)
    if k > 0:
        return jnp.concatenate([a[:, k:], f], axis=1)
    return jnp.concatenate([f, a[:, :l + k]], axis=1)


def _wshift(a, dw, w, fill):
    s = _shift_lanes(a, dw, fill)
    if dw == 0:
        return s
    wi = lax.rem(lax.broadcasted_iota(jnp.int32, a.shape, 1), jnp.int32(w))
    if dw > 0:
        valid = wi < (w - dw)
    else:
        valid = wi >= (-dw)
    return jnp.where(valid, s, fill)


def _accum_stats(st_ref, vals_list, ones_ref, pid):
    local = None
    for vals_bf in vals_list:
        both = jnp.concatenate([vals_bf, vals_bf * vals_bf], axis=0)
        d = jnp.dot(both, ones_ref[...], preferred_element_type=jnp.float32)
        local = d if local is None else local + d

    @pl.when(pid == 0)
    def _():
        st_ref[...] = local

    @pl.when(pid > 0)
    def _():
        st_ref[...] = st_ref[...] + local


def _fold(s_rows, q_rows, g_col, b_col, count):
    mean = s_rows[:, 0:1] / count
    var = q_rows[:, 0:1] / count - mean * mean
    scale = g_col * lax.rsqrt(var + _EPS)
    shift = b_col - mean * scale
    c = scale.shape[0]
    return (jnp.broadcast_to(scale, (c, 128)),
            jnp.broadcast_to(shift, (c, 128)))


def _col(*rows):
    return jnp.transpose(jnp.concatenate(rows, axis=1))


def _mega_body(x_ref, w1_ref, w2a_ref, w3a_ref, wp_ref, w3_ref, w5_ref,
               ones_ref,
               g1_ref, g4_ref, g2a_ref, g3a_ref, g2b_ref, g3b_ref,
               h1_ref, h4_ref, h2a_ref, h3a_ref, h2b_ref, h3b_ref,
               out_ref,
               keep_s, mid_s, acc2_s, st1_s, st2_s, scm_s, shm_s,
               sca_s, sha_s, wxb_s, wpb_s, w3b_s, w5b_s, *,
               n_imgs, blk, h, w, out1, red3, red5, out3):
    phase = pl.program_id(0)
    i = pl.program_id(1)
    count = float(n_imgs * h * w)
    cmid = red3 + red5
    ckeep = keep_s.shape[1]
    c1 = ckeep + cmid

    @pl.when(jnp.logical_and(phase == 0, i == 0))
    def _():
        wxb_s[...] = jnp.concatenate(
            [jnp.transpose(w1_ref[...]), jnp.transpose(w2a_ref[...]),
             jnp.transpose(w3a_ref[...])], axis=0).astype(jnp.bfloat16)
        wpb_s[...] = jnp.transpose(wp_ref[...]).astype(jnp.bfloat16)
        kk3 = w3_ref.shape[0] * w3_ref.shape[1] * w3_ref.shape[2]
        kk5 = w5_ref.shape[0] * w5_ref.shape[1] * w5_ref.shape[2]
        w3b_s[...] = jnp.transpose(
            w3_ref[...].reshape(kk3, w3_ref.shape[3])).astype(jnp.bfloat16)
        w5b_s[...] = jnp.transpose(
            w5_ref[...].reshape(kk5, w5_ref.shape[3])).astype(jnp.bfloat16)

    @pl.when(phase == 0)
    def _():
        cin = x_ref.shape[1]
        half = cin // 2
        vals = []
        for j in range(blk):
            y = jnp.dot(wxb_s[...], x_ref[j].astype(jnp.bfloat16),
                        preferred_element_type=jnp.float32)

            yp = None
            for cc in range(2):
                xc = x_ref[j, cc * half:(cc + 1) * half, :]
                rowm = jnp.maximum(xc, jnp.maximum(_wshift(xc, 1, w, _NEG),
                                                   _wshift(xc, -1, w, _NEG)))
                pooled = jnp.maximum(
                    rowm, jnp.maximum(_shift_lanes(rowm, w, _NEG),
                                      _shift_lanes(rowm, -w, _NEG)))
                z = jnp.dot(wpb_s[:, cc * half:(cc + 1) * half],
                            pooled.astype(jnp.bfloat16),
                            preferred_element_type=jnp.float32)
                yp = z if yp is None else yp + z

            keep_bf = jnp.concatenate([y[:out1], yp],
                                      axis=0).astype(jnp.bfloat16)
            mid_bf = y[out1:].astype(jnp.bfloat16)
            keep_s[blk * i + j] = keep_bf
            mid_s[blk * i + j] = mid_bf
            vals.append(jnp.concatenate([keep_bf, mid_bf], axis=0))
        _accum_stats(st1_s, vals, ones_ref, i)

    @pl.when(jnp.logical_and(phase == 1, i == 0))
    def _():
        sc, sh = _fold(st1_s[ckeep:c1], st1_s[c1 + ckeep:2 * c1],
                       _col(g2a_ref[...], g3a_ref[...]),
                       _col(h2a_ref[...], h3a_ref[...]), count)
        scm_s[...] = sc
        shm_s[...] = sh

    @pl.when(phase == 1)
    def _():
        vals = []
        for j in range(blk):
            mid = mid_s[blk * i + j].astype(jnp.float32)
            a = jnp.maximum(mid * scm_s[:, 0:1] + shm_s[:, 0:1], 0.0)
            r3 = a[:red3]
            r5 = a[red3:]

            xw3 = jnp.concatenate(
                [_wshift(r3, dw, w, 0.0) for dw in (-1, 0, 1)],
                axis=0).astype(jnp.bfloat16)
            y3 = _shift_lanes(
                jnp.dot(w3b_s[:, :3 * red3], xw3,
                        preferred_element_type=jnp.float32), -w, 0.0)
            y3 += jnp.dot(w3b_s[:, 3 * red3:6 * red3], xw3,
                          preferred_element_type=jnp.float32)
            y3 += _shift_lanes(
                jnp.dot(w3b_s[:, 6 * red3:], xw3,
                        preferred_element_type=jnp.float32), w, 0.0)

            xw5 = jnp.concatenate(
                [_wshift(r5, dw, w, 0.0) for dw in (-2, -1, 0, 1, 2)],
                axis=0).astype(jnp.bfloat16)
            y5 = 0.0
            for kh in range(5):
                z = jnp.dot(w5b_s[:, kh * 5 * red5:(kh + 1) * 5 * red5],
                            xw5, preferred_element_type=jnp.float32)
                y5 = y5 + _shift_lanes(z, (kh - 2) * w, 0.0)

            out2_bf = jnp.concatenate([y3, y5], axis=0).astype(jnp.bfloat16)
            acc2_s[blk * i + j] = out2_bf
            vals.append(out2_bf)
        _accum_stats(st2_s, vals, ones_ref, i)

    @pl.when(jnp.logical_and(phase == 2, i == 0))
    def _():
        c2 = acc2_s.shape[1]
        sck, shk = _fold(st1_s[:ckeep], st1_s[c1:c1 + ckeep],
                         _col(g1_ref[...], g4_ref[...]),
                         _col(h1_ref[...], h4_ref[...]), count)
        sc2, sh2 = _fold(st2_s[:c2], st2_s[c2:],
                         _col(g2b_ref[...], g3b_ref[...]),
                         _col(h2b_ref[...], h3b_ref[...]), count)
        sca_s[...] = jnp.concatenate([sck[:out1], sc2, sck[out1:]], axis=0)
        sha_s[...] = jnp.concatenate([shk[:out1], sh2, shk[out1:]], axis=0)

    @pl.when(phase == 2)
    def _():
        for j in range(blk):
            k = keep_s[blk * i + j]
            raw = jnp.concatenate([k[:out1], acc2_s[blk * i + j], k[out1:]],
                                  axis=0).astype(jnp.float32)
            res = jnp.maximum(raw * sca_s[:, 0:1] + sha_s[:, 0:1], 0.0)
            out_ref[j] = res.astype(jnp.bfloat16)


def kernel(x, b1_w, b1_gamma, b1_beta, b2a_w, b2a_gamma, b2a_beta,
           b2b_w, b2b_gamma, b2b_beta, b3a_w, b3a_gamma, b3a_beta,
           b3b_w, b3b_gamma, b3b_beta, b4_w, b4_gamma, b4_beta):
    x = x.astype(jnp.float32)
    n, cin, h, w = x.shape
    hw = h * w
    out1 = b1_w.shape[-1]
    red3, out3 = b2a_w.shape[-1], b2b_w.shape[-1]
    red5, out5 = b3a_w.shape[-1], b3b_w.shape[-1]
    outp = b4_w.shape[-1]
    cmid = red3 + red5
    ckeep = out1 + outp
    c2 = out3 + out5
    cout = out1 + out3 + out5 + outp

    w1_r = b1_w.reshape(cin, out1)
    w2a_r = b2a_w.reshape(cin, red3)
    w3a_r = b3a_w.reshape(cin, red5)
    wp_r = b4_w.reshape(cin, outp)

    ones_bf = jnp.ones((hw, 128), jnp.bfloat16)
    gb_vecs = [v.reshape(1, -1) for v in
               (b1_gamma, b4_gamma, b2a_gamma, b3a_gamma, b2b_gamma,
                b3b_gamma, b1_beta, b4_beta, b2a_beta, b3a_beta,
                b2b_beta, b3b_beta)]

    blk = next(b for b in (8, 4, 2, 1) if n % b == 0)
    last = n // blk - 1
    out = pl.pallas_call(
        functools.partial(_mega_body, n_imgs=n, blk=blk, h=h, w=w,
                          out1=out1, red3=red3, red5=red5, out3=out3),
        grid=(3, n // blk),
        in_specs=[
            pl.BlockSpec((blk, cin, hw),
                         lambda p, i: (jnp.where(p == 0, i, last), 0, 0)),
            pl.BlockSpec((cin, out1), lambda p, i: (0, 0)),
            pl.BlockSpec((cin, red3), lambda p, i: (0, 0)),
            pl.BlockSpec((cin, red5), lambda p, i: (0, 0)),
            pl.BlockSpec((cin, outp), lambda p, i: (0, 0)),
            pl.BlockSpec((3, 3, red3, out3), lambda p, i: (0, 0, 0, 0)),
            pl.BlockSpec((5, 5, red5, out5),
                         lambda p, i: (0, 0, 0, 0)),
            pl.BlockSpec((hw, 128), lambda p, i: (0, 0)),
        ] + [pl.BlockSpec(v.shape, lambda p, i: (0, 0)) for v in gb_vecs],
        out_specs=pl.BlockSpec(
            (blk, cout, hw),
            lambda p, i: (jnp.where(p == 2, i, 0), 0, 0)),
        out_shape=jax.ShapeDtypeStruct((n, cout, hw), jnp.bfloat16),
        scratch_shapes=[
            pltpu.VMEM((n, ckeep, hw), jnp.bfloat16),
            pltpu.VMEM((n, cmid, hw), jnp.bfloat16),
            pltpu.VMEM((n, c2, hw), jnp.bfloat16),
            pltpu.VMEM((2 * (ckeep + cmid), 128), jnp.float32),
            pltpu.VMEM((2 * c2, 128), jnp.float32),
            pltpu.VMEM((cmid, 128), jnp.float32),
            pltpu.VMEM((cmid, 128), jnp.float32),
            pltpu.VMEM((cout, 128), jnp.float32),
            pltpu.VMEM((cout, 128), jnp.float32),
            pltpu.VMEM((out1 + cmid, cin), jnp.bfloat16),
            pltpu.VMEM((outp, cin), jnp.bfloat16),
            pltpu.VMEM((out3, 9 * red3), jnp.bfloat16),
            pltpu.VMEM((out5, 25 * red5), jnp.bfloat16),
        ],
        compiler_params=pltpu.CompilerParams(
            dimension_semantics=("arbitrary", "arbitrary"),
            vmem_limit_bytes=100 * 1024 * 1024),
    )(x.reshape(n, cin, hw), w1_r, w2a_r, w3a_r, wp_r, b2b_w, b3b_w,
      ones_bf, *gb_vecs)
    return out.reshape(n, cout, h, w).astype(jnp.float32)
